# baseline (speedup 1.0000x reference)
import jax
import jax.numpy as jnp
from jax.experimental import pallas as pl
from jax.experimental.pallas import tpu as pltpu

N = 4096
D = 256
H = 64
BLK = 256
NBLK = N // BLK
K = 7
M0 = NBLK - K


def _body(s1_ref, s2_ref, adj_ref, raw_ref, w1_ref, b1_ref, a1_ref,
          w2_ref, b2_ref, a2_ref, wb_ref, bb_ref, wfc2_ref,
          np_ref, wnp_ref, ap_ref, wap_ref,
          emb_ref, resid_ref, sc1_ref, sc2_ref, logit_ref, npo_ref, apo_ref,
          stash_ref, ft_ref, h1_ref, x_ref, hsum_ref, v_ref):
    p = pl.program_id(0)
    j = pl.program_id(1)
    base = j * BLK

    @pl.when(p == 0)
    def _():
        f1 = jnp.dot(s1_ref[...], w1_ref[...],
                     preferred_element_type=jnp.float32)
        f2 = jnp.dot(s2_ref[...], w1_ref[...],
                     preferred_element_type=jnp.float32)
        ft_ref[pl.ds(base, BLK), :] = jnp.concatenate([f1, f2], axis=1)

        @pl.when(j == 0)
        def _():
            npo_ref[...] = jnp.dot(np_ref[...], wnp_ref[...],
                                   preferred_element_type=jnp.float32)
            apo_ref[...] = jnp.dot(ap_ref[...], wap_ref[...],
                                   preferred_element_type=jnp.float32)

    @pl.when(p == 1)
    def _():
        ablk = adj_ref[...]

        @pl.when(j >= M0)
        def _():
            stash_ref[pl.ds((j - M0) * BLK, BLK), :] = ablk

        acc = jnp.dot(ablk, ft_ref[...],
                      preferred_element_type=jnp.float32) + b1_ref[...]
        a = a1_ref[0, 0]
        h = jnp.where(acc >= 0, acc, a * acc)
        h1_ref[pl.ds(base, BLK), :] = h[:, :H]
        x1 = jnp.dot(h[:, :H], w2_ref[...], preferred_element_type=jnp.float32)
        x2 = jnp.dot(h[:, H:], w2_ref[...], preferred_element_type=jnp.float32)
        x_ref[pl.ds(base, BLK), :] = jnp.concatenate([x1, x2], axis=1)
        s = jnp.sum(h[:, :H], axis=0, keepdims=True)

        @pl.when(j == 0)
        def _():
            hsum_ref[...] = jnp.zeros_like(hsum_ref)
        hsum_ref[...] += s

    @pl.when(p == 2)
    def _():
        @pl.when(j == 0)
        def _():
            c = jax.nn.sigmoid(hsum_ref[...] * (1.0 / N))
            v_ref[...] = jnp.dot(wb_ref[...], c.T,
                                 preferred_element_type=jnp.float32)

        def tail(lhs):
            acc = jnp.dot(lhs, x_ref[...],
                          preferred_element_type=jnp.float32) + b2_ref[...]
            a = a2_ref[0, 0]
            o = jnp.where(acc >= 0, acc, a * acc)
            emb_ref[...] = o[:, :H]
            ft_ref[pl.ds(base, BLK), :] = jnp.concatenate(
                [o[:, :H], jnp.ones((BLK, H), jnp.float32)], axis=1)
            v = v_ref[...]
            bb = bb_ref[0, 0]
            s1v = jnp.dot(h1_ref[pl.ds(base, BLK), :], v,
                          preferred_element_type=jnp.float32) + bb
            s2v = jnp.dot(o[:, H:], v, preferred_element_type=jnp.float32) + bb
            sc1_ref[...] = jnp.broadcast_to(s1v, (BLK, H))
            sc2_ref[...] = jnp.broadcast_to(s2v, (BLK, H))

        @pl.when(j < M0)
        def _():
            tail(adj_ref[...])

        @pl.when(j >= M0)
        def _():
            tail(stash_ref[pl.ds((j - M0) * BLK, BLK), :])

    @pl.when(p == 3)
    def _():
        rblk = raw_ref[...]
        acc_all = jnp.dot(rblk, ft_ref[...],
                          preferred_element_type=jnp.float32)
        dpart = raw_ref[:, pl.ds(base, BLK)]
        rr = jax.lax.broadcasted_iota(jnp.int32, (BLK, BLK), 0)
        cc = jax.lax.broadcasted_iota(jnp.int32, (BLK, BLK), 1)
        d = jnp.sum(jnp.where(rr == cc, dpart, 0.0), axis=1, keepdims=True)
        emb_rows = ft_ref[pl.ds(base, BLK), :H]
        num = acc_all[:, :H] - d * emb_rows
        rs = acc_all[:, H:H + 1] - d
        safe = jnp.where(rs == 0.0, 1.0, rs)
        sub = jnp.where(rs == 0.0, 0.0, num / safe)
        resid = emb_rows - sub
        resid_ref[...] = resid
        logit_ref[...] = jnp.broadcast_to(
            jnp.dot(resid, wfc2_ref[...], preferred_element_type=jnp.float32),
            (BLK, H))


def kernel(seq1, seq2, adj, raw_adj, normal_prompt, abnormal_prompt,
           W1, b1, a1, W2, b2, a2, Wb, bb, Wfc2, Wnp, Wap):
    s1 = seq1.reshape(N, D)
    s2 = seq2.reshape(N, D)
    adj2 = adj.reshape(N, N)
    b1c = jnp.concatenate([b1, b1]).reshape(1, 2 * H)
    b2c = jnp.concatenate([b2, b2]).reshape(1, 2 * H)
    a1r = a1.reshape(1, 1)
    a2r = a2.reshape(1, 1)
    bbr = bb.reshape(1, 1)

    f32 = jnp.float32
    const = lambda shape: pl.BlockSpec(shape, lambda p, j: (0, 0))

    emb, resid, sc1, sc2, logit, np_out, ap_out = pl.pallas_call(
        _body,
        grid=(4, NBLK),
        in_specs=[
            pl.BlockSpec((BLK, D),
                         lambda p, j: (jnp.where(p == 0, j, NBLK - 1), 0)),
            pl.BlockSpec((BLK, D),
                         lambda p, j: (jnp.where(p == 0, j, NBLK - 1), 0)),
            pl.BlockSpec(
                (BLK, N),
                lambda p, j: (jnp.where(
                    p == 1, j,
                    jnp.where(p == 2, jnp.minimum(j, M0 - 1),
                              jnp.where(p == 0, 0, M0 - 1))), 0)),
            pl.BlockSpec((BLK, N),
                         lambda p, j: (jnp.where(p == 3, j, 0), 0)),
            const((D, H)), const((1, 2 * H)), const((1, 1)),
            const((H, H)), const((1, 2 * H)), const((1, 1)),
            const((H, H)), const((1, 1)), const((H, 1)),
            const((1, H)), const((H, H)), const((1, H)), const((H, H)),
        ],
        out_specs=[
            pl.BlockSpec(
                (BLK, H),
                lambda p, j: (jnp.where(p == 2, j,
                                        jnp.where(p == 3, NBLK - 1, 0)), 0)),
            pl.BlockSpec((BLK, H),
                         lambda p, j: (jnp.where(p == 3, j, 0), 0)),
            pl.BlockSpec(
                (BLK, H),
                lambda p, j: (jnp.where(p == 2, j,
                                        jnp.where(p == 3, NBLK - 1, 0)), 0)),
            pl.BlockSpec(
                (BLK, H),
                lambda p, j: (jnp.where(p == 2, j,
                                        jnp.where(p == 3, NBLK - 1, 0)), 0)),
            pl.BlockSpec((BLK, H),
                         lambda p, j: (jnp.where(p == 3, j, 0), 0)),
            const((1, H)), const((1, H)),
        ],
        out_shape=[
            jax.ShapeDtypeStruct((N, H), f32),
            jax.ShapeDtypeStruct((N, H), f32),
            jax.ShapeDtypeStruct((N, H), f32),
            jax.ShapeDtypeStruct((N, H), f32),
            jax.ShapeDtypeStruct((N, H), f32),
            jax.ShapeDtypeStruct((1, H), f32),
            jax.ShapeDtypeStruct((1, H), f32),
        ],
        scratch_shapes=[
            pltpu.VMEM((K * BLK, N), f32),
            pltpu.VMEM((N, 2 * H), f32),
            pltpu.VMEM((N, H), f32),
            pltpu.VMEM((N, 2 * H), f32),
            pltpu.VMEM((1, H), f32),
            pltpu.VMEM((H, 1), f32),
        ],
    )(s1, s2, adj2, raw_adj, W1, b1c, a1r, W2, b2c, a2r, Wb, bbr, Wfc2,
      normal_prompt, Wnp, abnormal_prompt, Wap)

    ret = jnp.concatenate([sc1[:, 0].reshape(1, N), sc2[:, 0].reshape(1, N)],
                          axis=1)
    return (ret, logit[:, :1][None], emb[None], resid[None], np_out, ap_out)


# speedup vs baseline: 1.0330x; 1.0330x over previous
import jax
import jax.numpy as jnp
from jax.experimental import pallas as pl
from jax.experimental.pallas import tpu as pltpu

N = 4096
D = 256
H = 64
BLK = 256
NBLK = N // BLK
K = 7
M0 = NBLK - K
_CN = (((0,), (1,)), ((), ()))


def _p2rowblk(j):
    return jnp.where(j < 2 * K, jnp.where(j % 2 == 0, j // 2, M0 + j // 2),
                     j - K)


def _p2hbmblk(j):
    return jnp.where(j < 2 * K, j // 2, j - K)


def _body(s1_ref, s2_ref, adj_ref, raw_ref, w1_ref, b1_ref, a1_ref,
          w2_ref, b2_ref, a2_ref, wb_ref, bb_ref, wfc2_ref,
          np_ref, wnp_ref, ap_ref, wap_ref,
          emb_ref, resid_ref, ret_ref, logit_ref, npo_ref, apo_ref,
          stash_ref, ft_ref, h1_ref, x_ref, hsum_ref, v_ref):
    p = pl.program_id(0)
    j = pl.program_id(1)
    base = j * BLK

    @pl.when(p == 0)
    def _():
        f1 = jnp.dot(s1_ref[...], w1_ref[...],
                     preferred_element_type=jnp.float32)
        f2 = jnp.dot(s2_ref[...], w1_ref[...],
                     preferred_element_type=jnp.float32)
        ft_ref[pl.ds(base, BLK), :] = jnp.concatenate([f1, f2], axis=1)

        @pl.when(j == 0)
        def _():
            npo_ref[...] = jnp.dot(np_ref[...], wnp_ref[...],
                                   preferred_element_type=jnp.float32)
            apo_ref[...] = jnp.dot(ap_ref[...], wap_ref[...],
                                   preferred_element_type=jnp.float32)

    @pl.when(p == 1)
    def _():
        ablk = adj_ref[...]

        @pl.when(j >= M0)
        def _():
            stash_ref[pl.ds((j - M0) * BLK, BLK), :] = ablk

        acc = jnp.dot(ablk, ft_ref[...],
                      preferred_element_type=jnp.float32) + b1_ref[...]
        a = a1_ref[0, 0]
        h = jnp.where(acc >= 0, acc, a * acc)
        h1_ref[pl.ds(base, BLK), :] = h[:, :H]
        x1 = jnp.dot(h[:, :H], w2_ref[...], preferred_element_type=jnp.float32)
        x2 = jnp.dot(h[:, H:], w2_ref[...], preferred_element_type=jnp.float32)
        x_ref[pl.ds(base, BLK), :] = jnp.concatenate([x1, x2], axis=1)
        s = jnp.sum(h[:, :H], axis=0, keepdims=True)

        @pl.when(j == 0)
        def _():
            hsum_ref[...] = jnp.zeros_like(hsum_ref)
        hsum_ref[...] += s

    @pl.when(p == 2)
    def _():
        @pl.when(j == 0)
        def _():
            c = jax.nn.sigmoid(hsum_ref[...] * (1.0 / N))
            v_ref[...] = jnp.dot(wb_ref[...], c.T,
                                 preferred_element_type=jnp.float32)

        def tail(lhs, rb):
            b2 = rb * BLK
            acc = jnp.dot(lhs, x_ref[...],
                          preferred_element_type=jnp.float32) + b2_ref[...]
            a = a2_ref[0, 0]
            o = jnp.where(acc >= 0, acc, a * acc)
            emb_ref[...] = o[:, :H]
            ft_ref[pl.ds(b2, BLK), :] = jnp.concatenate(
                [o[:, :H], jnp.ones((BLK, H), jnp.float32)], axis=1)
            v = v_ref[...]
            bb = bb_ref[0, 0]
            s1v = jax.lax.dot_general(
                v, h1_ref[pl.ds(b2, BLK), :], _CN,
                preferred_element_type=jnp.float32) + bb
            s2v = jax.lax.dot_general(
                v, o[:, H:], _CN, preferred_element_type=jnp.float32) + bb
            ret_ref[:, pl.ds(b2, BLK)] = s1v
            ret_ref[:, pl.ds(N + b2, BLK)] = s2v

        is_stash = jnp.logical_and(j < 2 * K, j % 2 == 1)

        @pl.when(jnp.logical_not(is_stash))
        def _():
            tail(adj_ref[...], _p2rowblk(j))

        @pl.when(is_stash)
        def _():
            tail(stash_ref[pl.ds((j // 2) * BLK, BLK), :], M0 + j // 2)

    @pl.when(p == 3)
    def _():
        rblk = raw_ref[...]
        acc_all = jnp.dot(rblk, ft_ref[...],
                          preferred_element_type=jnp.float32)
        dpart = raw_ref[:, pl.ds(base, BLK)]
        rr = jax.lax.broadcasted_iota(jnp.int32, (BLK, BLK), 0)
        cc = jax.lax.broadcasted_iota(jnp.int32, (BLK, BLK), 1)
        d = jnp.sum(jnp.where(rr == cc, dpart, 0.0), axis=1, keepdims=True)
        emb_rows = ft_ref[pl.ds(base, BLK), :H]
        num = acc_all[:, :H] - d * emb_rows
        rs = acc_all[:, H:H + 1] - d
        safe = jnp.where(rs == 0.0, 1.0, rs)
        sub = jnp.where(rs == 0.0, 0.0, num / safe)
        resid = emb_rows - sub
        resid_ref[...] = resid
        logit_ref[:, pl.ds(base, BLK)] = jax.lax.dot_general(
            wfc2_ref[...], resid, _CN, preferred_element_type=jnp.float32)


def kernel(seq1, seq2, adj, raw_adj, normal_prompt, abnormal_prompt,
           W1, b1, a1, W2, b2, a2, Wb, bb, Wfc2, Wnp, Wap):
    s1 = seq1.reshape(N, D)
    s2 = seq2.reshape(N, D)
    adj2 = adj.reshape(N, N)
    b1c = jnp.concatenate([b1, b1]).reshape(1, 2 * H)
    b2c = jnp.concatenate([b2, b2]).reshape(1, 2 * H)
    a1r = a1.reshape(1, 1)
    a2r = a2.reshape(1, 1)
    bbr = bb.reshape(1, 1)

    f32 = jnp.float32
    const = lambda shape: pl.BlockSpec(shape, lambda p, j: (0, 0))

    emb, resid, ret, logit, np_out, ap_out = pl.pallas_call(
        _body,
        grid=(4, NBLK),
        in_specs=[
            pl.BlockSpec((BLK, D),
                         lambda p, j: (jnp.where(p == 0, j, NBLK - 1), 0)),
            pl.BlockSpec((BLK, D),
                         lambda p, j: (jnp.where(p == 0, j, NBLK - 1), 0)),
            pl.BlockSpec(
                (BLK, N),
                lambda p, j: (jnp.where(
                    p == 1, j,
                    jnp.where(p == 2, _p2hbmblk(j),
                              jnp.where(p == 0, 0, M0 - 1))), 0)),
            pl.BlockSpec((BLK, N),
                         lambda p, j: (jnp.where(p == 3, j, 0), 0)),
            const((D, H)), const((1, 2 * H)), const((1, 1)),
            const((H, H)), const((1, 2 * H)), const((1, 1)),
            const((H, H)), const((1, 1)), const((H, 1)),
            const((1, H)), const((H, H)), const((1, H)), const((H, H)),
        ],
        out_specs=[
            pl.BlockSpec(
                (BLK, H),
                lambda p, j: (jnp.where(p == 2, _p2rowblk(j),
                                        jnp.where(p == 3, M0 - 1, 0)), 0)),
            pl.BlockSpec((BLK, H),
                         lambda p, j: (jnp.where(p == 3, j, 0), 0)),
            const((1, 2 * N)), const((1, N)),
            const((1, H)), const((1, H)),
        ],
        out_shape=[
            jax.ShapeDtypeStruct((N, H), f32),
            jax.ShapeDtypeStruct((N, H), f32),
            jax.ShapeDtypeStruct((1, 2 * N), f32),
            jax.ShapeDtypeStruct((1, N), f32),
            jax.ShapeDtypeStruct((1, H), f32),
            jax.ShapeDtypeStruct((1, H), f32),
        ],
        scratch_shapes=[
            pltpu.VMEM((K * BLK, N), f32),
            pltpu.VMEM((N, 2 * H), f32),
            pltpu.VMEM((N, H), f32),
            pltpu.VMEM((N, 2 * H), f32),
            pltpu.VMEM((1, H), f32),
            pltpu.VMEM((H, 1), f32),
        ],
    )(s1, s2, adj2, raw_adj, W1, b1c, a1r, W2, b2c, a2r, Wb, bbr, Wfc2,
      normal_prompt, Wnp, abnormal_prompt, Wap)

    return (ret, logit.reshape(1, N, 1), emb[None], resid[None],
            np_out, ap_out)


# speedup vs baseline: 1.2880x; 1.2469x over previous
import jax
import jax.numpy as jnp
from jax.experimental import pallas as pl
from jax.experimental.pallas import tpu as pltpu

N = 4096
D = 256
H = 64
BLK = 512
NBLK = N // BLK
_CN = (((0,), (1,)), ((), ()))


def _body(s1_ref, s2_ref, adj_ref, raw_ref, w1_ref, b1_ref, a1_ref,
          w2_ref, b2_ref, a2_ref, wb_ref, bb_ref, wfc2_ref,
          np_ref, wnp_ref, ap_ref, wap_ref,
          emb_ref, resid_ref, ret_ref, logit_ref, npo_ref, apo_ref,
          ft_ref, h1_ref, x_ref, hsum_ref, v_ref):
    p = pl.program_id(0)
    j = pl.program_id(1)
    base = j * BLK

    @pl.when(p == 0)
    def _():
        f1 = jnp.dot(s1_ref[...], w1_ref[...],
                     preferred_element_type=jnp.float32)
        f2 = jnp.dot(s2_ref[...], w1_ref[...],
                     preferred_element_type=jnp.float32)
        ft_ref[pl.ds(base, BLK), :] = jnp.concatenate([f1, f2], axis=1)

        @pl.when(j == 0)
        def _():
            npo_ref[...] = jnp.dot(np_ref[...], wnp_ref[...],
                                   preferred_element_type=jnp.float32)
            apo_ref[...] = jnp.dot(ap_ref[...], wap_ref[...],
                                   preferred_element_type=jnp.float32)

    @pl.when(p == 1)
    def _():
        acc = jnp.dot(adj_ref[...], ft_ref[...],
                      preferred_element_type=jnp.float32) + b1_ref[...]
        a = a1_ref[0, 0]
        h = jnp.where(acc >= 0, acc, a * acc)
        h1_ref[pl.ds(base, BLK), :] = h[:, :H]
        x1 = jnp.dot(h[:, :H], w2_ref[...], preferred_element_type=jnp.float32)
        x2 = jnp.dot(h[:, H:], w2_ref[...], preferred_element_type=jnp.float32)
        x_ref[pl.ds(base, BLK), :] = jnp.concatenate([x1, x2], axis=1)
        s = jnp.sum(h[:, :H], axis=0, keepdims=True)

        @pl.when(j == 0)
        def _():
            hsum_ref[...] = jnp.zeros_like(hsum_ref)
        hsum_ref[...] += s

    @pl.when(p == 2)
    def _():
        @pl.when(j == 0)
        def _():
            c = jax.nn.sigmoid(hsum_ref[...] * (1.0 / N))
            v_ref[...] = jnp.dot(wb_ref[...], c.T,
                                 preferred_element_type=jnp.float32)

        acc = jnp.dot(adj_ref[...], x_ref[...],
                      preferred_element_type=jnp.float32) + b2_ref[...]
        a = a2_ref[0, 0]
        o = jnp.where(acc >= 0, acc, a * acc)
        emb_ref[...] = o[:, :H]
        ft_ref[pl.ds(base, BLK), :] = jnp.concatenate(
            [o[:, :H], jnp.ones((BLK, H), jnp.float32)], axis=1)
        v = v_ref[...]
        bb = bb_ref[0, 0]
        s1v = jax.lax.dot_general(
            v, h1_ref[pl.ds(base, BLK), :], _CN,
            preferred_element_type=jnp.float32) + bb
        s2v = jax.lax.dot_general(
            v, o[:, H:], _CN, preferred_element_type=jnp.float32) + bb
        ret_ref[:, pl.ds(base, BLK)] = s1v
        ret_ref[:, pl.ds(N + base, BLK)] = s2v

    @pl.when(p == 3)
    def _():
        rblk = raw_ref[...]
        acc_all = jnp.dot(rblk, ft_ref[...],
                          preferred_element_type=jnp.float32)
        dpart = raw_ref[:, pl.ds(base, BLK)]
        rr = jax.lax.broadcasted_iota(jnp.int32, (BLK, BLK), 0)
        cc = jax.lax.broadcasted_iota(jnp.int32, (BLK, BLK), 1)
        d = jnp.sum(jnp.where(rr == cc, dpart, 0.0), axis=1, keepdims=True)
        emb_rows = ft_ref[pl.ds(base, BLK), :H]
        num = acc_all[:, :H] - d * emb_rows
        rs = acc_all[:, H:H + 1] - d
        safe = jnp.where(rs == 0.0, 1.0, rs)
        sub = jnp.where(rs == 0.0, 0.0, num / safe)
        resid = emb_rows - sub
        resid_ref[...] = resid
        logit_ref[:, pl.ds(base, BLK)] = jax.lax.dot_general(
            wfc2_ref[...], resid, _CN, preferred_element_type=jnp.float32)


def kernel(seq1, seq2, adj, raw_adj, normal_prompt, abnormal_prompt,
           W1, b1, a1, W2, b2, a2, Wb, bb, Wfc2, Wnp, Wap):
    s1 = seq1.reshape(N, D)
    s2 = seq2.reshape(N, D)
    adj2 = adj.reshape(N, N)
    b1c = jnp.concatenate([b1, b1]).reshape(1, 2 * H)
    b2c = jnp.concatenate([b2, b2]).reshape(1, 2 * H)
    a1r = a1.reshape(1, 1)
    a2r = a2.reshape(1, 1)
    bbr = bb.reshape(1, 1)

    f32 = jnp.float32
    const = lambda shape: pl.BlockSpec(shape, lambda p, j: (0, 0))

    emb, resid, ret, logit, np_out, ap_out = pl.pallas_call(
        _body,
        grid=(4, NBLK),
        in_specs=[
            pl.BlockSpec((BLK, D),
                         lambda p, j: (jnp.where(p == 0, j, NBLK - 1), 0)),
            pl.BlockSpec((BLK, D),
                         lambda p, j: (jnp.where(p == 0, j, NBLK - 1), 0)),
            pl.BlockSpec(
                (BLK, N),
                lambda p, j: (jnp.where(
                    p == 0, 0, jnp.where(p == 3, NBLK - 1, j)), 0)),
            pl.BlockSpec((BLK, N),
                         lambda p, j: (jnp.where(p == 3, j, 0), 0)),
            const((D, H)), const((1, 2 * H)), const((1, 1)),
            const((H, H)), const((1, 2 * H)), const((1, 1)),
            const((H, H)), const((1, 1)), const((H, 1)),
            const((1, H)), const((H, H)), const((1, H)), const((H, H)),
        ],
        out_specs=[
            pl.BlockSpec(
                (BLK, H),
                lambda p, j: (jnp.where(p == 2, j,
                                        jnp.where(p == 3, NBLK - 1, 0)), 0)),
            pl.BlockSpec((BLK, H),
                         lambda p, j: (jnp.where(p == 3, j, 0), 0)),
            const((1, 2 * N)), const((1, N)),
            const((1, H)), const((1, H)),
        ],
        out_shape=[
            jax.ShapeDtypeStruct((N, H), f32),
            jax.ShapeDtypeStruct((N, H), f32),
            jax.ShapeDtypeStruct((1, 2 * N), f32),
            jax.ShapeDtypeStruct((1, N), f32),
            jax.ShapeDtypeStruct((1, H), f32),
            jax.ShapeDtypeStruct((1, H), f32),
        ],
        scratch_shapes=[
            pltpu.VMEM((N, 2 * H), f32),
            pltpu.VMEM((N, H), f32),
            pltpu.VMEM((N, 2 * H), f32),
            pltpu.VMEM((1, H), f32),
            pltpu.VMEM((H, 1), f32),
        ],
    )(s1, s2, adj2, raw_adj, W1, b1c, a1r, W2, b2c, a2r, Wb, bbr, Wfc2,
      normal_prompt, Wnp, abnormal_prompt, Wap)

    return (ret, logit.reshape(1, N, 1), emb[None], resid[None],
            np_out, ap_out)


# speedup vs baseline: 1.3061x; 1.0140x over previous
import jax
import jax.numpy as jnp
from jax.experimental import pallas as pl
from jax.experimental.pallas import tpu as pltpu

N = 4096
D = 256
H = 64
BLK = 512
NBLK = N // BLK
_CN = (((0,), (1,)), ((), ()))


def _body(s1_ref, s2_ref, adj_ref, raw_ref, w1_ref, b1_ref, a1_ref,
          w2_ref, b2_ref, a2_ref, wb_ref, bb_ref, wfc2_ref,
          np_ref, wnp_ref, ap_ref, wap_ref,
          emb_ref, resid_ref, ret_ref, logit_ref, npo_ref, apo_ref,
          ft_ref, h1_ref, x_ref, hsum_ref, v_ref):
    p = pl.program_id(0)
    j = pl.program_id(1)
    base = j * BLK

    @pl.when(p == 0)
    def _():
        f1 = jnp.dot(s1_ref[...], w1_ref[...],
                     preferred_element_type=jnp.float32)
        f2 = jnp.dot(s2_ref[...], w1_ref[...],
                     preferred_element_type=jnp.float32)
        ft_ref[pl.ds(base, BLK), :] = jnp.concatenate([f1, f2], axis=1)

        @pl.when(j == 0)
        def _():
            npo_ref[...] = jnp.dot(np_ref[...], wnp_ref[...],
                                   preferred_element_type=jnp.float32)
            apo_ref[...] = jnp.dot(ap_ref[...], wap_ref[...],
                                   preferred_element_type=jnp.float32)

    @pl.when(p == 1)
    def _():
        b1c = jnp.concatenate([b1_ref[...], b1_ref[...]], axis=1)
        acc = jnp.dot(adj_ref[...], ft_ref[...],
                      preferred_element_type=jnp.float32) + b1c
        a = a1_ref[0, 0]
        h = jnp.where(acc >= 0, acc, a * acc)
        h1_ref[pl.ds(base, BLK), :] = h[:, :H]
        x1 = jnp.dot(h[:, :H], w2_ref[...], preferred_element_type=jnp.float32)
        x2 = jnp.dot(h[:, H:], w2_ref[...], preferred_element_type=jnp.float32)
        x_ref[pl.ds(base, BLK), :] = jnp.concatenate([x1, x2], axis=1)
        s = jnp.sum(h[:, :H], axis=0, keepdims=True)

        @pl.when(j == 0)
        def _():
            hsum_ref[...] = jnp.zeros_like(hsum_ref)
        hsum_ref[...] += s

    @pl.when(p == 2)
    def _():
        @pl.when(j == 0)
        def _():
            c = jax.nn.sigmoid(hsum_ref[...] * (1.0 / N))
            v_ref[...] = jnp.dot(wb_ref[...], c.T,
                                 preferred_element_type=jnp.float32)

        b2c = jnp.concatenate([b2_ref[...], b2_ref[...]], axis=1)
        acc = jnp.dot(adj_ref[...], x_ref[...],
                      preferred_element_type=jnp.float32) + b2c
        a = a2_ref[0, 0]
        o = jnp.where(acc >= 0, acc, a * acc)
        emb_ref[...] = o[:, :H]
        ft_ref[pl.ds(base, BLK), :] = jnp.concatenate(
            [o[:, :H], jnp.ones((BLK, H), jnp.float32)], axis=1)
        v = v_ref[...]
        bb = bb_ref[0, 0]
        s1v = jax.lax.dot_general(
            v, h1_ref[pl.ds(base, BLK), :], _CN,
            preferred_element_type=jnp.float32) + bb
        s2v = jax.lax.dot_general(
            v, o[:, H:], _CN, preferred_element_type=jnp.float32) + bb
        ret_ref[:, pl.ds(base, BLK)] = s1v
        ret_ref[:, pl.ds(N + base, BLK)] = s2v

    @pl.when(p == 3)
    def _():
        rblk = raw_ref[...]
        acc_all = jnp.dot(rblk, ft_ref[...],
                          preferred_element_type=jnp.float32)
        dpart = raw_ref[:, pl.ds(base, BLK)]
        rr = jax.lax.broadcasted_iota(jnp.int32, (BLK, BLK), 0)
        cc = jax.lax.broadcasted_iota(jnp.int32, (BLK, BLK), 1)
        d = jnp.sum(jnp.where(rr == cc, dpart, 0.0), axis=1, keepdims=True)
        emb_rows = ft_ref[pl.ds(base, BLK), :H]
        num = acc_all[:, :H] - d * emb_rows
        rs = acc_all[:, H:H + 1] - d
        safe = jnp.where(rs == 0.0, 1.0, rs)
        sub = jnp.where(rs == 0.0, 0.0, num / safe)
        resid = emb_rows - sub
        resid_ref[...] = resid
        logit_ref[:, pl.ds(base, BLK)] = jax.lax.dot_general(
            wfc2_ref[...], resid, _CN, preferred_element_type=jnp.float32)


def kernel(seq1, seq2, adj, raw_adj, normal_prompt, abnormal_prompt,
           W1, b1, a1, W2, b2, a2, Wb, bb, Wfc2, Wnp, Wap):
    s1 = seq1.reshape(N, D)
    s2 = seq2.reshape(N, D)
    adj2 = adj.reshape(N, N)
    b1r = b1.reshape(1, H)
    b2r = b2.reshape(1, H)
    a1r = a1.reshape(1, 1)
    a2r = a2.reshape(1, 1)
    bbr = bb.reshape(1, 1)

    f32 = jnp.float32
    const = lambda shape: pl.BlockSpec(shape, lambda p, j: (0, 0))

    emb, resid, ret, logit, np_out, ap_out = pl.pallas_call(
        _body,
        grid=(4, NBLK),
        in_specs=[
            pl.BlockSpec((BLK, D),
                         lambda p, j: (jnp.where(p == 0, j, NBLK - 1), 0)),
            pl.BlockSpec((BLK, D),
                         lambda p, j: (jnp.where(p == 0, j, NBLK - 1), 0)),
            pl.BlockSpec(
                (BLK, N),
                lambda p, j: (jnp.where(
                    p == 0, 0, jnp.where(p == 3, NBLK - 1, j)), 0)),
            pl.BlockSpec((BLK, N),
                         lambda p, j: (jnp.where(p == 3, j, 0), 0)),
            const((D, H)), const((1, H)), const((1, 1)),
            const((H, H)), const((1, H)), const((1, 1)),
            const((H, H)), const((1, 1)), const((H, 1)),
            const((1, H)), const((H, H)), const((1, H)), const((H, H)),
        ],
        out_specs=[
            pl.BlockSpec(
                (BLK, H),
                lambda p, j: (jnp.where(p == 2, j,
                                        jnp.where(p == 3, NBLK - 1, 0)), 0)),
            pl.BlockSpec((BLK, H),
                         lambda p, j: (jnp.where(p == 3, j, 0), 0)),
            const((1, 2 * N)), const((1, N)),
            const((1, H)), const((1, H)),
        ],
        out_shape=[
            jax.ShapeDtypeStruct((N, H), f32),
            jax.ShapeDtypeStruct((N, H), f32),
            jax.ShapeDtypeStruct((1, 2 * N), f32),
            jax.ShapeDtypeStruct((1, N), f32),
            jax.ShapeDtypeStruct((1, H), f32),
            jax.ShapeDtypeStruct((1, H), f32),
        ],
        scratch_shapes=[
            pltpu.VMEM((N, 2 * H), f32),
            pltpu.VMEM((N, H), f32),
            pltpu.VMEM((N, 2 * H), f32),
            pltpu.VMEM((1, H), f32),
            pltpu.VMEM((H, 1), f32),
        ],
    )(s1, s2, adj2, raw_adj, W1, b1r, a1r, W2, b2r, a2r, Wb, bbr, Wfc2,
      normal_prompt, Wnp, abnormal_prompt, Wap)

    return (ret, logit.reshape(1, N, 1), emb[None], resid[None],
            np_out, ap_out)


# speedup vs baseline: 1.3246x; 1.0142x over previous
import jax
import jax.numpy as jnp
from jax.experimental import pallas as pl
from jax.experimental.pallas import tpu as pltpu

N = 4096
D = 256
H = 64
BLK = 512
NBLK = N // BLK
KS = 2
M0 = NBLK - KS
_CN = (((0,), (1,)), ((), ()))


def _p2rowblk(j):
    return jnp.where(j < 2 * KS,
                     jnp.where(j % 2 == 0, j // 2, M0 + j // 2), j - KS)


def _p2hbmblk(j):
    return jnp.where(j < 2 * KS, j // 2, j - KS)


def _body(s1_ref, s2_ref, adj_ref, raw_ref, w1_ref, b1_ref, a1_ref,
          w2_ref, b2_ref, a2_ref, wb_ref, bb_ref, wfc2_ref,
          np_ref, wnp_ref, ap_ref, wap_ref,
          emb_ref, resid_ref, ret_ref, logit_ref, npo_ref, apo_ref,
          stash_ref, ft_ref, h1_ref, x_ref, hsum_ref, v_ref):
    p = pl.program_id(0)
    j = pl.program_id(1)
    base = j * BLK

    @pl.when(p == 0)
    def _():
        f1 = jnp.dot(s1_ref[...], w1_ref[...],
                     preferred_element_type=jnp.float32)
        f2 = jnp.dot(s2_ref[...], w1_ref[...],
                     preferred_element_type=jnp.float32)
        ft_ref[pl.ds(base, BLK), :] = jnp.concatenate([f1, f2], axis=1)

        @pl.when(j == 0)
        def _():
            npo_ref[...] = jnp.dot(np_ref[...], wnp_ref[...],
                                   preferred_element_type=jnp.float32)
            apo_ref[...] = jnp.dot(ap_ref[...], wap_ref[...],
                                   preferred_element_type=jnp.float32)

    @pl.when(p == 1)
    def _():
        @pl.when(j >= M0)
        def _():
            stash_ref[pl.ds((j - M0) * BLK, BLK), :] = adj_ref[...]

        b1c = jnp.concatenate([b1_ref[...], b1_ref[...]], axis=1)
        acc = jnp.dot(adj_ref[...], ft_ref[...],
                      preferred_element_type=jnp.float32) + b1c
        a = a1_ref[0, 0]
        h = jnp.where(acc >= 0, acc, a * acc)
        h1_ref[pl.ds(base, BLK), :] = h[:, :H]
        x1 = jnp.dot(h[:, :H], w2_ref[...], preferred_element_type=jnp.float32)
        x2 = jnp.dot(h[:, H:], w2_ref[...], preferred_element_type=jnp.float32)
        x_ref[pl.ds(base, BLK), :] = jnp.concatenate([x1, x2], axis=1)
        s = jnp.sum(h[:, :H], axis=0, keepdims=True)

        @pl.when(j == 0)
        def _():
            hsum_ref[...] = jnp.zeros_like(hsum_ref)
        hsum_ref[...] += s

    @pl.when(p == 2)
    def _():
        @pl.when(j == 0)
        def _():
            c = jax.nn.sigmoid(hsum_ref[...] * (1.0 / N))
            v_ref[...] = jnp.dot(wb_ref[...], c.T,
                                 preferred_element_type=jnp.float32)

        def tail(lhs, rb):
            rbase = rb * BLK
            b2c = jnp.concatenate([b2_ref[...], b2_ref[...]], axis=1)
            acc = jnp.dot(lhs, x_ref[...],
                          preferred_element_type=jnp.float32) + b2c
            a = a2_ref[0, 0]
            o = jnp.where(acc >= 0, acc, a * acc)
            emb_ref[...] = o[:, :H]
            ft_ref[pl.ds(rbase, BLK), :] = jnp.concatenate(
                [o[:, :H], jnp.ones((BLK, H), jnp.float32)], axis=1)
            v = v_ref[...]
            bb = bb_ref[0, 0]
            s1v = jax.lax.dot_general(
                v, h1_ref[pl.ds(rbase, BLK), :], _CN,
                preferred_element_type=jnp.float32) + bb
            s2v = jax.lax.dot_general(
                v, o[:, H:], _CN, preferred_element_type=jnp.float32) + bb
            ret_ref[:, pl.ds(rbase, BLK)] = s1v
            ret_ref[:, pl.ds(N + rbase, BLK)] = s2v

        is_stash = jnp.logical_and(j < 2 * KS, j % 2 == 1)

        @pl.when(jnp.logical_not(is_stash))
        def _():
            tail(adj_ref[...], _p2rowblk(j))

        @pl.when(is_stash)
        def _():
            tail(stash_ref[pl.ds((j // 2) * BLK, BLK), :], M0 + j // 2)

    @pl.when(p == 3)
    def _():
        rblk = raw_ref[...]
        acc_all = jnp.dot(rblk, ft_ref[...],
                          preferred_element_type=jnp.float32)
        dpart = raw_ref[:, pl.ds(base, BLK)]
        rr = jax.lax.broadcasted_iota(jnp.int32, (BLK, BLK), 0)
        cc = jax.lax.broadcasted_iota(jnp.int32, (BLK, BLK), 1)
        d = jnp.sum(jnp.where(rr == cc, dpart, 0.0), axis=1, keepdims=True)
        emb_rows = ft_ref[pl.ds(base, BLK), :H]
        num = acc_all[:, :H] - d * emb_rows
        rs = acc_all[:, H:H + 1] - d
        safe = jnp.where(rs == 0.0, 1.0, rs)
        sub = jnp.where(rs == 0.0, 0.0, num / safe)
        resid = emb_rows - sub
        resid_ref[...] = resid
        logit_ref[:, pl.ds(base, BLK)] = jax.lax.dot_general(
            wfc2_ref[...], resid, _CN, preferred_element_type=jnp.float32)


def kernel(seq1, seq2, adj, raw_adj, normal_prompt, abnormal_prompt,
           W1, b1, a1, W2, b2, a2, Wb, bb, Wfc2, Wnp, Wap):
    s1 = seq1.reshape(N, D)
    s2 = seq2.reshape(N, D)
    adj2 = adj.reshape(N, N)
    b1r = b1.reshape(1, H)
    b2r = b2.reshape(1, H)
    a1r = a1.reshape(1, 1)
    a2r = a2.reshape(1, 1)
    bbr = bb.reshape(1, 1)

    f32 = jnp.float32
    const = lambda shape: pl.BlockSpec(shape, lambda p, j: (0, 0))

    emb, resid, ret, logit, np_out, ap_out = pl.pallas_call(
        _body,
        grid=(4, NBLK),
        in_specs=[
            pl.BlockSpec((BLK, D),
                         lambda p, j: (jnp.where(p == 0, j, NBLK - 1), 0)),
            pl.BlockSpec((BLK, D),
                         lambda p, j: (jnp.where(p == 0, j, NBLK - 1), 0)),
            pl.BlockSpec(
                (BLK, N),
                lambda p, j: (jnp.where(
                    p == 0, 0,
                    jnp.where(p == 1, j,
                              jnp.where(p == 2, _p2hbmblk(j), M0 - 1))), 0)),
            pl.BlockSpec((BLK, N),
                         lambda p, j: (jnp.where(p == 3, j, 0), 0)),
            const((D, H)), const((1, H)), const((1, 1)),
            const((H, H)), const((1, H)), const((1, 1)),
            const((H, H)), const((1, 1)), const((H, 1)),
            const((1, H)), const((H, H)), const((1, H)), const((H, H)),
        ],
        out_specs=[
            pl.BlockSpec(
                (BLK, H),
                lambda p, j: (jnp.where(p == 2, _p2rowblk(j),
                                        jnp.where(p == 3, M0 - 1, 0)), 0)),
            pl.BlockSpec((BLK, H),
                         lambda p, j: (jnp.where(p == 3, j, 0), 0)),
            const((1, 2 * N)), const((1, N)),
            const((1, H)), const((1, H)),
        ],
        out_shape=[
            jax.ShapeDtypeStruct((N, H), f32),
            jax.ShapeDtypeStruct((N, H), f32),
            jax.ShapeDtypeStruct((1, 2 * N), f32),
            jax.ShapeDtypeStruct((1, N), f32),
            jax.ShapeDtypeStruct((1, H), f32),
            jax.ShapeDtypeStruct((1, H), f32),
        ],
        scratch_shapes=[
            pltpu.VMEM((KS * BLK, N), f32),
            pltpu.VMEM((N, 2 * H), f32),
            pltpu.VMEM((N, H), f32),
            pltpu.VMEM((N, 2 * H), f32),
            pltpu.VMEM((1, H), f32),
            pltpu.VMEM((H, 1), f32),
        ],
    )(s1, s2, adj2, raw_adj, W1, b1r, a1r, W2, b2r, a2r, Wb, bbr, Wfc2,
      normal_prompt, Wnp, abnormal_prompt, Wap)

    return (ret, logit.reshape(1, N, 1), emb[None], resid[None],
            np_out, ap_out)
